# baseline (speedup 1.0000x reference)
import functools
import jax
import jax.numpy as jnp
from jax import lax
from jax.experimental import pallas as pl
from jax.experimental.pallas import tpu as pltpu
from jax.experimental.pallas import tpu_sc as plsc

QB = 256
KB = 256
HB = 8

_MASKED = -1e30
_MINIT = -1e29


def _ranges_on_sc(docs):
    n = docs.shape[0]
    nq = n // QB
    info = plsc.get_sparse_core_info()
    nc = info.num_cores

    mesh = plsc.VectorSubcoreMesh(core_axis_name="c", subcore_axis_name="s")

    @functools.partial(
        pl.kernel, mesh=mesh,
        out_type=jax.ShapeDtypeStruct((nq, 16), jnp.int32),
        scratch_types=[
            pltpu.VMEM((n,), jnp.int32),
            pltpu.VMEM((16,), jnp.int32),
        ],
    )
    def ranges_kernel(docs_hbm, out_hbm, docs_v, row_v):
        wid = lax.axis_index("s") * nc + lax.axis_index("c")

        dnums = lax.GatherDimensionNumbers(
            offset_dims=(), collapsed_slice_dims=(0,), start_index_map=(0,))

        def _perm(vec, idx):
            return lax.gather(vec, idx.reshape(16, 1), dnums, slice_sizes=(1,),
                              mode=lax.GatherScatterMode.PROMISE_IN_BOUNDS)

        def _splat_lane(vec, lane):
            return _perm(vec, jnp.full((16,), lane, jnp.int32))

        def _lanesum(vec):
            lanes = lax.iota(jnp.int32, 16)
            for shift in (8, 4, 2, 1):
                vec = vec + _perm(vec, (lanes + shift) % 16)
            return vec

        @pl.when(wid < nq)
        def _():
            pltpu.sync_copy(docs_hbm, docs_v)
            first = docs_v[pl.ds(wid * QB, 16)]
            last = docs_v[pl.ds(wid * QB + QB - 16, 16)]
            qmin = _splat_lane(first, 0)
            qmax = _splat_lane(last, 15)
            zero = jnp.zeros((16,), jnp.int32)
            one = jnp.ones((16,), jnp.int32)

            def body(j, carry):
                lo_acc, hi_acc = carry
                c = docs_v[pl.ds(j * 16, 16)]
                lo_acc = lo_acc + jnp.where(c < qmin, one, zero)
                hi_acc = hi_acc + jnp.where(c <= qmax, one, zero)
                return lo_acc, hi_acc

            lo_acc, hi_acc = lax.fori_loop(0, n // 16, body, (zero, zero))
            lane = lax.iota(jnp.int32, 16)
            row_v[...] = jnp.where(lane == 0, _lanesum(lo_acc), _lanesum(hi_acc))
            pltpu.sync_copy(row_v, out_hbm.at[wid])

    return ranges_kernel(docs)


def _flash_kernel(ranges_ref, docs_col_ref, docs_row_ref, q_ref, k_ref, v_ref,
                  o_ref):
    d = q_ref.shape[-1]
    i = pl.program_id(1)
    kb_lo = ranges_ref[2 * i] // KB
    kb_hi = (ranges_ref[2 * i + 1] + KB - 1) // KB
    docs_q = docs_col_ref[:, 0:1]

    scale = 1.0 / (d ** 0.5)
    qs = [q_ref[hh] for hh in range(HB)]

    def body(kb, carry):
        off = kb * KB
        docs_k = docs_row_ref[0:1, pl.ds(off, KB)]
        mask = docs_q == docs_k
        out = []
        for hh in range(HB):
            m, l, acc = carry[hh]
            k = k_ref[hh, pl.ds(off, KB), :]
            v = v_ref[hh, pl.ds(off, KB), :]
            s = jax.lax.dot_general(qs[hh], k, (((1,), (1,)), ((), ())),
                                    preferred_element_type=jnp.float32) * scale
            s = jnp.where(mask, s, _MASKED)
            m_new = jnp.maximum(m, jnp.max(s, axis=1, keepdims=True))
            p = jnp.exp(s - m_new)
            alpha = jnp.exp(m - m_new)
            l_new = l * alpha + jnp.sum(p, axis=1, keepdims=True)
            acc_new = acc * alpha + jax.lax.dot_general(
                p.astype(jnp.bfloat16), v, (((1,), (0,)), ((), ())),
                preferred_element_type=jnp.float32)
            out.append((m_new, l_new, acc_new))
        return tuple(out)

    init = tuple((jnp.full((QB, 1), _MINIT, jnp.float32),
                  jnp.zeros((QB, 1), jnp.float32),
                  jnp.zeros((QB, d), jnp.float32)) for _ in range(HB))
    final = lax.fori_loop(kb_lo, kb_hi, body, init)
    for hh in range(HB):
        m, l, acc = final[hh]
        o_ref[hh] = acc / l


def kernel(Q, K, V, document_id):
    b, h, n, d = Q.shape
    docs = document_id.astype(jnp.int32)
    Qr = Q.reshape(b * h, n, d).astype(jnp.bfloat16)
    Kr = K.reshape(b * h, n, d).astype(jnp.bfloat16)
    Vr = V.reshape(b * h, n, d).astype(jnp.bfloat16)
    docs_col = jnp.broadcast_to(docs[:, None], (n, 8))
    docs_row = jnp.broadcast_to(docs[None, :], (8, n))

    nq = n // QB
    ranges = _ranges_on_sc(docs)[:, :2].reshape(-1)

    grid_spec = pltpu.PrefetchScalarGridSpec(
        num_scalar_prefetch=1,
        grid=(b * h // HB, nq),
        in_specs=[
            pl.BlockSpec((QB, 8), lambda g, i, r: (i, 0)),
            pl.BlockSpec((8, n), lambda g, i, r: (0, 0)),
            pl.BlockSpec((HB, QB, d), lambda g, i, r: (g, i, 0)),
            pl.BlockSpec((HB, n, d), lambda g, i, r: (g, 0, 0)),
            pl.BlockSpec((HB, n, d), lambda g, i, r: (g, 0, 0)),
        ],
        out_specs=pl.BlockSpec((HB, QB, d), lambda g, i, r: (g, i, 0)),
    )
    out = pl.pallas_call(
        _flash_kernel,
        grid_spec=grid_spec,
        out_shape=jax.ShapeDtypeStruct((b * h, n, d), jnp.float32),
        compiler_params=pltpu.CompilerParams(
            dimension_semantics=("parallel", "arbitrary")),
    )(ranges, docs_col, docs_row, Qr, Kr, Vr)
    return out.reshape(b, h, n, d)


# speedup vs baseline: 1.1846x; 1.1846x over previous
import functools
import jax
import jax.numpy as jnp
from jax import lax
from jax.experimental import pallas as pl
from jax.experimental.pallas import tpu as pltpu
from jax.experimental.pallas import tpu_sc as plsc

QB = 256
KB = 256
HB = 16

_MASKED = -1e30
_MINIT = -1e29


def _ranges_on_sc(docs):
    n = docs.shape[0]
    nq = n // QB
    info = plsc.get_sparse_core_info()
    nc = info.num_cores

    mesh = plsc.VectorSubcoreMesh(core_axis_name="c", subcore_axis_name="s")

    @functools.partial(
        pl.kernel, mesh=mesh,
        out_type=jax.ShapeDtypeStruct((nq, 16), jnp.int32),
        scratch_types=[
            pltpu.VMEM((n,), jnp.int32),
            pltpu.VMEM((16,), jnp.int32),
        ],
    )
    def ranges_kernel(docs_hbm, out_hbm, docs_v, row_v):
        wid = lax.axis_index("s") * nc + lax.axis_index("c")

        dnums = lax.GatherDimensionNumbers(
            offset_dims=(), collapsed_slice_dims=(0,), start_index_map=(0,))

        def _perm(vec, idx):
            return lax.gather(vec, idx.reshape(16, 1), dnums, slice_sizes=(1,),
                              mode=lax.GatherScatterMode.PROMISE_IN_BOUNDS)

        def _splat_lane(vec, lane):
            return _perm(vec, jnp.full((16,), lane, jnp.int32))

        def _lanesum(vec):
            lanes = lax.iota(jnp.int32, 16)
            for shift in (8, 4, 2, 1):
                vec = vec + _perm(vec, (lanes + shift) % 16)
            return vec

        @pl.when(wid < nq)
        def _():
            pltpu.sync_copy(docs_hbm, docs_v)
            first = docs_v[pl.ds(wid * QB, 16)]
            last = docs_v[pl.ds(wid * QB + QB - 16, 16)]
            qmin = _splat_lane(first, 0)
            qmax = _splat_lane(last, 15)
            zero = jnp.zeros((16,), jnp.int32)
            one = jnp.ones((16,), jnp.int32)

            def body(j, carry):
                lo_acc, hi_acc = carry
                c = docs_v[pl.ds(j * 16, 16)]
                lo_acc = lo_acc + jnp.where(c < qmin, one, zero)
                hi_acc = hi_acc + jnp.where(c <= qmax, one, zero)
                return lo_acc, hi_acc

            lo_acc, hi_acc = lax.fori_loop(0, n // 16, body, (zero, zero))
            lane = lax.iota(jnp.int32, 16)
            row_v[...] = jnp.where(lane == 0, _lanesum(lo_acc), _lanesum(hi_acc))
            pltpu.sync_copy(row_v, out_hbm.at[wid])

    return ranges_kernel(docs)


def _flash_kernel(ranges_ref, docs_col_ref, docs_row_ref, q_ref, k_ref, v_ref,
                  o_ref):
    d = q_ref.shape[-1]
    i = pl.program_id(1)
    kb_lo = ranges_ref[2 * i] // KB
    kb_hi = (ranges_ref[2 * i + 1] + KB - 1) // KB
    docs_q = docs_col_ref[:, 0:1]

    scale = 1.0 / (d ** 0.5)
    qs = [q_ref[hh] for hh in range(HB)]

    def body(kb, carry):
        off = kb * KB
        docs_k = docs_row_ref[0:1, pl.ds(off, KB)]
        mask = docs_q == docs_k
        out = []
        for hh in range(HB):
            m, l, acc = carry[hh]
            k = k_ref[hh, pl.ds(off, KB), :]
            v = v_ref[hh, pl.ds(off, KB), :]
            s = jax.lax.dot_general(qs[hh], k, (((1,), (1,)), ((), ())),
                                    preferred_element_type=jnp.float32) * scale
            s = jnp.where(mask, s, _MASKED)
            m_new = jnp.maximum(m, jnp.max(s, axis=1, keepdims=True))
            p = jnp.exp(s - m_new)
            alpha = jnp.exp(m - m_new)
            l_new = l * alpha + jnp.sum(p, axis=1, keepdims=True)
            acc_new = acc * alpha + jax.lax.dot_general(
                p, v, (((1,), (0,)), ((), ())),
                preferred_element_type=jnp.float32)
            out.append((m_new, l_new, acc_new))
        return tuple(out)

    init = tuple((jnp.full((QB, 1), _MINIT, jnp.float32),
                  jnp.zeros((QB, 1), jnp.float32),
                  jnp.zeros((QB, d), jnp.float32)) for _ in range(HB))
    final = lax.fori_loop(kb_lo, kb_hi, body, init)
    for hh in range(HB):
        m, l, acc = final[hh]
        o_ref[hh] = acc / l


def kernel(Q, K, V, document_id):
    b, h, n, d = Q.shape
    docs = document_id.astype(jnp.int32)
    Qr = Q.reshape(b * h, n, d)
    Kr = K.reshape(b * h, n, d)
    Vr = V.reshape(b * h, n, d)
    docs_col = jnp.broadcast_to(docs[:, None], (n, 8))
    docs_row = jnp.broadcast_to(docs[None, :], (8, n))

    nq = n // QB
    ranges = _ranges_on_sc(docs)[:, :2].reshape(-1)

    grid_spec = pltpu.PrefetchScalarGridSpec(
        num_scalar_prefetch=1,
        grid=(b * h // HB, nq),
        in_specs=[
            pl.BlockSpec((QB, 8), lambda g, i, r: (i, 0)),
            pl.BlockSpec((8, n), lambda g, i, r: (0, 0)),
            pl.BlockSpec((HB, QB, d), lambda g, i, r: (g, i, 0)),
            pl.BlockSpec((HB, n, d), lambda g, i, r: (g, 0, 0)),
            pl.BlockSpec((HB, n, d), lambda g, i, r: (g, 0, 0)),
        ],
        out_specs=pl.BlockSpec((HB, QB, d), lambda g, i, r: (g, i, 0)),
    )
    out = pl.pallas_call(
        _flash_kernel,
        grid_spec=grid_spec,
        out_shape=jax.ShapeDtypeStruct((b * h, n, d), jnp.float32),
        compiler_params=pltpu.CompilerParams(
            dimension_semantics=("parallel", "arbitrary")),
    )(ranges, docs_col, docs_row, Qr, Kr, Vr)
    return out.reshape(b, h, n, d)
